# baseline (speedup 1.0000x reference)
import jax
import jax.numpy as jnp
from jax.experimental import pallas as pl
from jax.experimental.pallas import tpu as pltpu

_CT_COLS = 200
_CT_ROWS = 1000
_IND_COLS = 250
_IND_ROWS = 400
_SLOTS_PER_BLK = 4


def _click_body(s_ref, ct_ref, ind_ref, t_ref, ct_out, ind_out):
    i = pl.program_id(0)
    idx = s_ref[0]

    ct_out[...] = ct_ref[...]

    @pl.when(i == 0)
    def _():
        off_r = idx // _IND_COLS
        off_c = idx % _IND_COLS
        row_i = jax.lax.broadcasted_iota(jnp.int32, (_IND_ROWS, _IND_COLS), 0)
        col_i = jax.lax.broadcasted_iota(jnp.int32, (_IND_ROWS, _IND_COLS), 1)
        hit = (row_i == off_r) & (col_i == off_c)
        ind_out[...] = ind_ref[...] + hit.astype(jnp.int32)

    @pl.when(i == idx // _CT_ROWS)
    def _():
        off_r = idx // _IND_COLS
        off_c = idx % _IND_COLS
        row_i = jax.lax.broadcasted_iota(jnp.int32, (_IND_ROWS, _IND_COLS), 0)
        col_i = jax.lax.broadcasted_iota(jnp.int32, (_IND_ROWS, _IND_COLS), 1)
        cursor = jnp.sum(jnp.where((row_i == off_r) & (col_i == off_c),
                                   ind_ref[...], 0))
        rr = idx - i * _CT_ROWS
        row_c = jax.lax.broadcasted_iota(jnp.int32, (_CT_ROWS, _CT_COLS), 0)
        col_c = jax.lax.broadcasted_iota(jnp.int32, (_CT_ROWS, _CT_COLS), 1)
        ct_out[...] = jnp.where((row_c == rr) & (col_c == cursor),
                                t_ref[0], ct_ref[...])


def _saved_body(s_ref, saved_ref, new_ref, out_ref):
    i = pl.program_id(0)
    si = s_ref[0]

    out_ref[...] = saved_ref[...]

    @pl.when(i == si // _SLOTS_PER_BLK)
    def _():
        local = jax.lax.broadcasted_iota(
            jnp.int32, (_SLOTS_PER_BLK,) + saved_ref.shape[1:], 0)
        out_ref[...] = jnp.where(local == si - i * _SLOTS_PER_BLK,
                                 new_ref[...], saved_ref[...])


def kernel(clicktimes, indices, idx, t, saved, new, save_index):
    idx32 = jnp.asarray(idx, jnp.int32).reshape(1)
    si32 = jnp.asarray(save_index, jnp.int32).reshape(1)
    t_arr = jnp.asarray(t, jnp.float32).reshape(1)
    ind2d = indices.reshape(_IND_ROWS, _IND_COLS)

    n_ct_blocks = clicktimes.shape[0] // _CT_ROWS

    ct_out, ind2d_out = pl.pallas_call(
        _click_body,
        grid_spec=pltpu.PrefetchScalarGridSpec(
            num_scalar_prefetch=1,
            grid=(n_ct_blocks,),
            in_specs=[
                pl.BlockSpec((_CT_ROWS, _CT_COLS), lambda i, s: (i, 0)),
                pl.BlockSpec((_IND_ROWS, _IND_COLS), lambda i, s: (0, 0)),
                pl.BlockSpec(memory_space=pltpu.SMEM),
            ],
            out_specs=[
                pl.BlockSpec((_CT_ROWS, _CT_COLS), lambda i, s: (i, 0)),
                pl.BlockSpec((_IND_ROWS, _IND_COLS), lambda i, s: (0, 0)),
            ],
        ),
        out_shape=[
            jax.ShapeDtypeStruct(clicktimes.shape, clicktimes.dtype),
            jax.ShapeDtypeStruct(ind2d.shape, ind2d.dtype),
        ],
        compiler_params=pltpu.CompilerParams(
            dimension_semantics=("arbitrary",)),
    )(idx32, clicktimes, ind2d, t_arr)

    n_sv_blocks = saved.shape[0] // _SLOTS_PER_BLK
    blk = (_SLOTS_PER_BLK,) + saved.shape[1:]
    saved_out = pl.pallas_call(
        _saved_body,
        grid_spec=pltpu.PrefetchScalarGridSpec(
            num_scalar_prefetch=1,
            grid=(n_sv_blocks,),
            in_specs=[
                pl.BlockSpec(blk, lambda i, s: (i, 0, 0)),
                pl.BlockSpec(blk, lambda i, s: (s[0] // _SLOTS_PER_BLK, 0, 0)),
            ],
            out_specs=pl.BlockSpec(blk, lambda i, s: (i, 0, 0)),
        ),
        out_shape=jax.ShapeDtypeStruct(saved.shape, saved.dtype),
        compiler_params=pltpu.CompilerParams(
            dimension_semantics=("arbitrary",)),
    )(si32, saved, new)

    return (ct_out, ind2d_out.reshape(indices.shape), saved_out,
            save_index + 1)
